# baseline (speedup 1.0000x reference)
import functools

import jax
import jax.numpy as jnp
from jax import lax
from jax.experimental import pallas as pl
from jax.experimental.pallas import tpu as pltpu
from jax.experimental.pallas import tpu_sc as plsc


_ROWS = 128
_EPAD = 128


def _router_body(x_ref, gw_ref, pos_ref, offs_ref):
    logits = lax.dot_general(
        gw_ref[...], x_ref[...], (((1,), (1,)), ((), ())),
        preferred_element_type=jnp.float32)
    e_num, s_tot = logits.shape
    mx = jnp.max(logits, axis=0, keepdims=True)
    ids = lax.broadcasted_iota(jnp.int32, logits.shape, 0)
    sel = jnp.min(jnp.where(logits == mx, ids, e_num), axis=0, keepdims=True)

    e_ids = lax.broadcasted_iota(jnp.int32, (_EPAD, s_tot), 0)
    m = sel == e_ids
    mf = m.astype(jnp.float32)

    blk = 128
    u_blk = (lax.broadcasted_iota(jnp.int32, (blk, blk), 0)
             < lax.broadcasted_iota(jnp.int32, (blk, blk), 1)
             ).astype(jnp.float32)
    n_blk = s_tot // blk
    u_nb = (lax.broadcasted_iota(jnp.int32, (n_blk, n_blk), 0)
            < lax.broadcasted_iota(jnp.int32, (n_blk, n_blk), 1)
            ).astype(jnp.float32)

    ones_row = jnp.ones((1, s_tot), jnp.float32)
    counts_row = lax.dot_general(
        ones_row, mf, (((1,), (1,)), ((), ())),
        preferred_element_type=jnp.float32)
    offs_row = lax.dot_general(
        counts_row, u_blk, (((1,), (0,)), ((), ())),
        precision=lax.Precision.HIGHEST,
        preferred_element_type=jnp.float32)

    mf64 = mf[:e_num]
    w_within = jnp.concatenate(
        [lax.dot_general(mf64[:, b * blk:(b + 1) * blk], u_blk,
                         (((1,), (0,)), ((), ())),
                         preferred_element_type=jnp.float32)
         for b in range(n_blk)], axis=1)
    bt = (lax.broadcasted_iota(jnp.int32, (n_blk, s_tot), 0)
          == lax.broadcasted_iota(jnp.int32, (n_blk, s_tot), 1) // blk
          ).astype(jnp.float32)
    p_eb = lax.dot_general(
        mf64, bt, (((1,), (1,)), ((), ())),
        preferred_element_type=jnp.float32)
    pc_eb = lax.dot_general(
        p_eb, u_nb, (((1,), (0,)), ((), ())),
        precision=lax.Precision.HIGHEST,
        preferred_element_type=jnp.float32)
    prior = lax.dot_general(
        pc_eb, bt, (((1,), (0,)), ((), ())),
        precision=lax.Precision.HIGHEST,
        preferred_element_type=jnp.float32)
    rank_row = jnp.sum(mf64 * (prior + w_within), axis=0, keepdims=True)

    offs_sel = lax.dot_general(
        offs_row, mf, (((1,), (0,)), ((), ())),
        precision=lax.Precision.HIGHEST,
        preferred_element_type=jnp.float32)

    pos_ref[...] = (offs_sel + rank_row).astype(jnp.int32)
    offs_ref[...] = offs_row.astype(jnp.int32)


def _make_sc_permute(n_rows, d, scatter):
    info = plsc.get_sparse_core_info()
    nw = info.num_cores * info.num_subcores
    b_per_w = n_rows // nw
    mesh = plsc.VectorSubcoreMesh(core_axis_name="c", subcore_axis_name="s")

    @functools.partial(
        pl.kernel, mesh=mesh,
        out_type=jax.ShapeDtypeStruct((n_rows, d), jnp.float32),
        scratch_types=[
            pltpu.VMEM((b_per_w,), jnp.int32),
            pltpu.VMEM((b_per_w, d), jnp.float32),
            pltpu.SemaphoreType.DMA,
        ],
    )
    def permute_kernel(table_hbm, idx_hbm, out_hbm, idx_v, rows_v, sem):
        wid = lax.axis_index("s") * info.num_cores + lax.axis_index("c")
        base = wid * b_per_w
        pltpu.sync_copy(idx_hbm.at[pl.ds(base, b_per_w)], idx_v)
        if scatter:
            pltpu.sync_copy(table_hbm.at[pl.ds(base, b_per_w)], rows_v)
            pltpu.async_copy(rows_v, out_hbm.at[idx_v], sem).wait()
        else:
            pltpu.async_copy(table_hbm.at[idx_v], rows_v, sem).wait()
            pltpu.sync_copy(rows_v, out_hbm.at[pl.ds(base, b_per_w)])

    return permute_kernel


def _moe_body(off_ref, pos_ref, x_ref, w1_ref, w3_ref, w2_ref, out_ref, *,
              seq):
    e = pl.program_id(0)
    start = off_ref[e]
    end = off_ref[e + 1]
    base0 = (start // 8) * 8
    nch = (end - base0 + _ROWS - 1) // _ROWS

    def chunk(i, carry):
        base = jnp.minimum(base0 + i * _ROWS, seq - _ROWS)
        base = pl.multiple_of(base, 8)
        r_ids = base + lax.broadcasted_iota(jnp.int32, (_ROWS, 1), 0)
        p_onehot = (pos_ref[...] == r_ids).astype(jnp.float32)
        xg = lax.dot_general(
            p_onehot, x_ref[...], (((1,), (0,)), ((), ())),
            preferred_element_type=jnp.float32)
        a = lax.dot_general(
            xg, w1_ref[0], (((1,), (1,)), ((), ())),
            preferred_element_type=jnp.float32)
        b = lax.dot_general(
            xg, w3_ref[0], (((1,), (1,)), ((), ())),
            preferred_element_type=jnp.float32)
        h = jnp.maximum(a, 0.0) * b
        o = lax.dot_general(
            h, w2_ref[0], (((1,), (1,)), ((), ())),
            preferred_element_type=jnp.float32)

        rows = base + lax.broadcasted_iota(jnp.int32, (_ROWS, 1), 0)
        mask = (rows >= start) & (rows < end)
        cur = out_ref[pl.ds(base, _ROWS), :]
        out_ref[pl.ds(base, _ROWS), :] = jnp.where(mask, o, cur)
        return carry

    jax.lax.fori_loop(0, nch, chunk, 0)


def kernel(x, gate_w, w1, w2, w3):
    bz, seq, d = x.shape
    e_num, ff, _ = w1.shape
    s_tot = bz * seq
    xt = x.reshape(s_tot, d)

    pos2d, offs2d = pl.pallas_call(
        _router_body,
        out_shape=[
            jax.ShapeDtypeStruct((1, s_tot), jnp.int32),
            jax.ShapeDtypeStruct((1, _EPAD), jnp.int32),
        ],
    )(xt, gate_w)
    pos = pos2d.reshape(s_tot)
    offs = offs2d.reshape(_EPAD)

    os_sorted = pl.pallas_call(
        functools.partial(_moe_body, seq=s_tot),
        grid=(e_num,),
        in_specs=[
            pl.BlockSpec(memory_space=pltpu.SMEM),
            pl.BlockSpec((1, s_tot), lambda e: (0, 0)),
            pl.BlockSpec((s_tot, d), lambda e: (0, 0)),
            pl.BlockSpec((1, ff, d), lambda e: (e, 0, 0)),
            pl.BlockSpec((1, ff, d), lambda e: (e, 0, 0)),
            pl.BlockSpec((1, d, ff), lambda e: (e, 0, 0)),
        ],
        out_specs=pl.BlockSpec((s_tot, d), lambda e: (0, 0)),
        out_shape=jax.ShapeDtypeStruct((s_tot, d), jnp.float32),
    )(offs, pos2d, xt, w1, w3, w2)

    sc_gather = _make_sc_permute(s_tot, d, scatter=False)
    out = sc_gather(os_sorted, pos)

    return out.reshape(bz, seq, d)


# speedup vs baseline: 1.0313x; 1.0313x over previous
import functools

import jax
import jax.numpy as jnp
from jax import lax
from jax.experimental import pallas as pl
from jax.experimental.pallas import tpu as pltpu
from jax.experimental.pallas import tpu_sc as plsc


_ROWS = 64
_EPAD = 128


def _router_body(x_ref, gw_ref, pos_ref, offs_ref):
    logits = lax.dot_general(
        gw_ref[...], x_ref[...], (((1,), (1,)), ((), ())),
        preferred_element_type=jnp.float32)
    e_num, s_tot = logits.shape
    mx = jnp.max(logits, axis=0, keepdims=True)
    ids = lax.broadcasted_iota(jnp.int32, logits.shape, 0)
    sel = jnp.min(jnp.where(logits == mx, ids, e_num), axis=0, keepdims=True)

    e_ids = lax.broadcasted_iota(jnp.int32, (_EPAD, s_tot), 0)
    m = sel == e_ids
    mf = m.astype(jnp.float32)

    blk = 128
    u_blk = (lax.broadcasted_iota(jnp.int32, (blk, blk), 0)
             < lax.broadcasted_iota(jnp.int32, (blk, blk), 1)
             ).astype(jnp.float32)
    n_blk = s_tot // blk
    u_nb = (lax.broadcasted_iota(jnp.int32, (n_blk, n_blk), 0)
            < lax.broadcasted_iota(jnp.int32, (n_blk, n_blk), 1)
            ).astype(jnp.float32)

    ones_row = jnp.ones((1, s_tot), jnp.float32)
    counts_row = lax.dot_general(
        ones_row, mf, (((1,), (1,)), ((), ())),
        preferred_element_type=jnp.float32)
    offs_row = lax.dot_general(
        counts_row, u_blk, (((1,), (0,)), ((), ())),
        precision=lax.Precision.HIGHEST,
        preferred_element_type=jnp.float32)

    mf64 = mf[:e_num]
    w_within = jnp.concatenate(
        [lax.dot_general(mf64[:, b * blk:(b + 1) * blk], u_blk,
                         (((1,), (0,)), ((), ())),
                         preferred_element_type=jnp.float32)
         for b in range(n_blk)], axis=1)
    bt = (lax.broadcasted_iota(jnp.int32, (n_blk, s_tot), 0)
          == lax.broadcasted_iota(jnp.int32, (n_blk, s_tot), 1) // blk
          ).astype(jnp.float32)
    p_eb = lax.dot_general(
        mf64, bt, (((1,), (1,)), ((), ())),
        preferred_element_type=jnp.float32)
    pc_eb = lax.dot_general(
        p_eb, u_nb, (((1,), (0,)), ((), ())),
        precision=lax.Precision.HIGHEST,
        preferred_element_type=jnp.float32)
    prior = lax.dot_general(
        pc_eb, bt, (((1,), (0,)), ((), ())),
        precision=lax.Precision.HIGHEST,
        preferred_element_type=jnp.float32)
    rank_row = jnp.sum(mf64 * (prior + w_within), axis=0, keepdims=True)

    offs_sel = lax.dot_general(
        offs_row, mf, (((1,), (0,)), ((), ())),
        precision=lax.Precision.HIGHEST,
        preferred_element_type=jnp.float32)

    pos_ref[...] = (offs_sel + rank_row).astype(jnp.int32)
    offs_ref[...] = offs_row.astype(jnp.int32)


def _make_sc_permute(n_rows, d, scatter):
    info = plsc.get_sparse_core_info()
    nw = info.num_cores * info.num_subcores
    b_per_w = n_rows // nw
    mesh = plsc.VectorSubcoreMesh(core_axis_name="c", subcore_axis_name="s")

    @functools.partial(
        pl.kernel, mesh=mesh,
        out_type=jax.ShapeDtypeStruct((n_rows, d), jnp.float32),
        scratch_types=[
            pltpu.VMEM((b_per_w,), jnp.int32),
            pltpu.VMEM((b_per_w, d), jnp.float32),
            pltpu.SemaphoreType.DMA,
        ],
    )
    def permute_kernel(table_hbm, idx_hbm, out_hbm, idx_v, rows_v, sem):
        wid = lax.axis_index("s") * info.num_cores + lax.axis_index("c")
        base = wid * b_per_w
        pltpu.sync_copy(idx_hbm.at[pl.ds(base, b_per_w)], idx_v)
        if scatter:
            pltpu.sync_copy(table_hbm.at[pl.ds(base, b_per_w)], rows_v)
            pltpu.async_copy(rows_v, out_hbm.at[idx_v], sem).wait()
        else:
            pltpu.async_copy(table_hbm.at[idx_v], rows_v, sem).wait()
            pltpu.sync_copy(rows_v, out_hbm.at[pl.ds(base, b_per_w)])

    return permute_kernel


def _moe_body(off_ref, xs_ref, w1_ref, w3_ref, w2_ref, out_ref, *, seq):
    e = pl.program_id(0)
    start = off_ref[e]
    end = off_ref[e + 1]
    base0 = (start // 8) * 8
    nch = (end - base0 + _ROWS - 1) // _ROWS

    def chunk(i, carry):
        base = jnp.minimum(base0 + i * _ROWS, seq - _ROWS)
        base = pl.multiple_of(base, 8)
        xg = xs_ref[pl.ds(base, _ROWS), :]
        a = lax.dot_general(
            xg, w1_ref[0], (((1,), (1,)), ((), ())),
            preferred_element_type=jnp.float32)
        b = lax.dot_general(
            xg, w3_ref[0], (((1,), (1,)), ((), ())),
            preferred_element_type=jnp.float32)
        h = jnp.maximum(a, 0.0) * b
        o = lax.dot_general(
            h, w2_ref[0], (((1,), (1,)), ((), ())),
            preferred_element_type=jnp.float32)

        rows = base + lax.broadcasted_iota(jnp.int32, (_ROWS, 1), 0)
        mask = (rows >= start) & (rows < end)
        cur = out_ref[pl.ds(base, _ROWS), :]
        out_ref[pl.ds(base, _ROWS), :] = jnp.where(mask, o, cur)
        return carry

    jax.lax.fori_loop(0, nch, chunk, 0)


def kernel(x, gate_w, w1, w2, w3):
    bz, seq, d = x.shape
    e_num, ff, _ = w1.shape
    s_tot = bz * seq
    xt = x.reshape(s_tot, d)

    pos2d, offs2d = pl.pallas_call(
        _router_body,
        out_shape=[
            jax.ShapeDtypeStruct((1, s_tot), jnp.int32),
            jax.ShapeDtypeStruct((1, _EPAD), jnp.int32),
        ],
    )(xt, gate_w)
    pos = pos2d.reshape(s_tot)
    offs = offs2d.reshape(_EPAD)

    sc_scatter = _make_sc_permute(s_tot, d, scatter=True)
    xs = sc_scatter(xt, pos)

    os_sorted = pl.pallas_call(
        functools.partial(_moe_body, seq=s_tot),
        grid=(e_num,),
        in_specs=[
            pl.BlockSpec(memory_space=pltpu.SMEM),
            pl.BlockSpec((s_tot, d), lambda e: (0, 0)),
            pl.BlockSpec((1, ff, d), lambda e: (e, 0, 0)),
            pl.BlockSpec((1, ff, d), lambda e: (e, 0, 0)),
            pl.BlockSpec((1, d, ff), lambda e: (e, 0, 0)),
        ],
        out_specs=pl.BlockSpec((s_tot, d), lambda e: (0, 0)),
        out_shape=jax.ShapeDtypeStruct((s_tot, d), jnp.float32),
    )(offs, xs, w1, w3, w2)

    sc_gather = _make_sc_permute(s_tot, d, scatter=False)
    out = sc_gather(os_sorted, pos)

    return out.reshape(bz, seq, d)
